# baseline (speedup 1.0000x reference)
import functools

import jax
import jax.numpy as jnp
from jax import lax
from jax.experimental import pallas as pl
from jax.experimental.pallas import tpu as pltpu
from jax.experimental.pallas import tpu_sc as plsc

VOCAB = 1000000
EMBED = 64
LABEL = 16
B = 4096
L = 200

NUM_CORES = 2
NUM_SUBCORES = 16
NW = NUM_CORES * NUM_SUBCORES
ROWS_PER_W = B // NW
CHUNK = 100
CHUNKS_PER_ROW = L // CHUNK
IDX_ROWS_PER_W = ROWS_PER_W * CHUNKS_PER_ROW
LANES = 16
EGROUPS = EMBED // LANES


def _pool_body(idx_hbm, table_hbm, out_hbm, idx_v, rows_v, pooled_v, sem):
    wid = lax.axis_index("s") * NUM_CORES + lax.axis_index("c")
    base = wid * ROWS_PER_W

    pltpu.sync_copy(idx_hbm.at[pl.ds(wid * IDX_ROWS_PER_W, IDX_ROWS_PER_W)],
                    idx_v)

    def example_body(e, carry):
        cp0 = pltpu.async_copy(table_hbm.at[idx_v.at[2 * e]],
                               rows_v.at[pl.ds(0, CHUNK)], sem)
        cp1 = pltpu.async_copy(table_hbm.at[idx_v.at[2 * e + 1]],
                               rows_v.at[pl.ds(CHUNK, CHUNK)], sem)
        cp0.wait()
        cp1.wait()

        def red(j, accs):
            new = list(accs)
            for c in range(EGROUPS):
                v = rows_v[j, pl.ds(c * LANES, LANES)]
                new[c] = accs[c] + v
                new[EGROUPS + c] = jnp.maximum(accs[EGROUPS + c], v)
            return tuple(new)

        init = tuple(
            [jnp.zeros((LANES,), jnp.float32)] * EGROUPS
            + [jnp.full((LANES,), -jnp.inf, jnp.float32)] * EGROUPS)
        accs = lax.fori_loop(0, L, red, init)

        inv_l = jnp.float32(1.0 / L)
        for c in range(EGROUPS):
            pooled_v[e, pl.ds(c * LANES, LANES)] = (
                accs[c] * inv_l + accs[EGROUPS + c])
        return carry

    lax.fori_loop(0, ROWS_PER_W, example_body, 0)
    pltpu.sync_copy(pooled_v, out_hbm.at[pl.ds(base, ROWS_PER_W)])


_pool_sc = functools.partial(
    pl.kernel,
    out_type=jax.ShapeDtypeStruct((B, EMBED), jnp.float32),
    mesh=plsc.VectorSubcoreMesh(core_axis_name="c", subcore_axis_name="s"),
    scratch_types=[
        pltpu.VMEM((IDX_ROWS_PER_W, CHUNK), jnp.int32),
        pltpu.VMEM((L, EMBED), jnp.float32),
        pltpu.VMEM((ROWS_PER_W, EMBED), jnp.float32),
        pltpu.SemaphoreType.DMA,
    ],
    compiler_params=pltpu.CompilerParams(use_tc_tiling_on_sc=False),
)(_pool_body)


def _mm_body(p_ref, w_ref, b_ref, o_ref):
    o_ref[...] = (
        jnp.dot(p_ref[...], w_ref[...], preferred_element_type=jnp.float32)
        + b_ref[...])


def _project(pooled, W, b):
    return pl.pallas_call(
        _mm_body,
        out_shape=jax.ShapeDtypeStruct((B, LABEL), jnp.float32),
    )(pooled, W, b.reshape(1, LABEL))


def kernel(input, emb_table, W, b):
    idx = input.astype(jnp.int32).reshape(B * L // CHUNK, CHUNK)
    pooled = _pool_sc(idx, emb_table)
    return _project(pooled, W, b)


# speedup vs baseline: 1.1174x; 1.1174x over previous
import functools

import jax
import jax.numpy as jnp
from jax import lax
from jax.experimental import pallas as pl
from jax.experimental.pallas import tpu as pltpu
from jax.experimental.pallas import tpu_sc as plsc

VOCAB = 1000000
EMBED = 64
LABEL = 16
B = 4096
L = 200

NUM_CORES = 2
NUM_SUBCORES = 16
NW = NUM_CORES * NUM_SUBCORES
ROWS_PER_W = B // NW
CHUNK = 100
CHUNKS_PER_ROW = L // CHUNK
IDX_ROWS_PER_W = ROWS_PER_W * CHUNKS_PER_ROW
LANES = 16
EGROUPS = EMBED // LANES


def _reduce_chunk(rows_ref, accs):
    def red(j, accs):
        sA = list(accs[0:4]); sB = list(accs[4:8])
        mA = list(accs[8:12]); mB = list(accs[12:16])
        for c in range(EGROUPS):
            va = rows_ref[2 * j, pl.ds(c * LANES, LANES)]
            vb = rows_ref[2 * j + 1, pl.ds(c * LANES, LANES)]
            sA[c] = sA[c] + va
            sB[c] = sB[c] + vb
            mA[c] = jnp.maximum(mA[c], va)
            mB[c] = jnp.maximum(mB[c], vb)
        return tuple(sA + sB + mA + mB)

    return lax.fori_loop(0, CHUNK // 2, red, accs, unroll=2)


def _fresh_accs():
    z = jnp.zeros((LANES,), jnp.float32)
    ninf = jnp.full((LANES,), -jnp.inf, jnp.float32)
    return tuple([z] * (2 * EGROUPS) + [ninf] * (2 * EGROUPS))


def _pool_body(idx_hbm, table_hbm, out_hbm, idx_v, rows0, rows1, pooled_v,
               sem0, sem1):
    wid = lax.axis_index("s") * NUM_CORES + lax.axis_index("c")
    base = wid * ROWS_PER_W

    pltpu.sync_copy(idx_hbm.at[pl.ds(wid * IDX_ROWS_PER_W, IDX_ROWS_PER_W)],
                    idx_v)

    pltpu.async_copy(table_hbm.at[idx_v.at[0]], rows0, sem0)

    def example_body(e, carry):
        cp1 = pltpu.async_copy(table_hbm.at[idx_v.at[2 * e + 1]], rows1, sem1)

        pltpu.make_async_copy(table_hbm.at[idx_v.at[2 * e]], rows0,
                              sem0).wait()
        accs = _reduce_chunk(rows0, _fresh_accs())

        nxt = jnp.minimum(2 * e + 2, IDX_ROWS_PER_W - 1)
        pltpu.async_copy(table_hbm.at[idx_v.at[nxt]], rows0, sem0)

        cp1.wait()
        accs = _reduce_chunk(rows1, accs)

        inv_l = jnp.float32(1.0 / L)
        for c in range(EGROUPS):
            s = accs[c] + accs[EGROUPS + c]
            m = jnp.maximum(accs[2 * EGROUPS + c], accs[3 * EGROUPS + c])
            pooled_v[e, pl.ds(c * LANES, LANES)] = s * inv_l + m
        return carry

    lax.fori_loop(0, ROWS_PER_W, example_body, 0)
    pltpu.make_async_copy(table_hbm.at[idx_v.at[0]], rows0, sem0).wait()
    pltpu.sync_copy(pooled_v, out_hbm.at[pl.ds(base, ROWS_PER_W)])


_pool_sc = functools.partial(
    pl.kernel,
    out_type=jax.ShapeDtypeStruct((B, EMBED), jnp.float32),
    mesh=plsc.VectorSubcoreMesh(core_axis_name="c", subcore_axis_name="s"),
    scratch_types=[
        pltpu.VMEM((IDX_ROWS_PER_W, CHUNK), jnp.int32),
        pltpu.VMEM((CHUNK, EMBED), jnp.float32),
        pltpu.VMEM((CHUNK, EMBED), jnp.float32),
        pltpu.VMEM((ROWS_PER_W, EMBED), jnp.float32),
        pltpu.SemaphoreType.DMA,
        pltpu.SemaphoreType.DMA,
    ],
    compiler_params=pltpu.CompilerParams(use_tc_tiling_on_sc=False),
)(_pool_body)


def _mm_body(p_ref, w_ref, b_ref, o_ref):
    o_ref[...] = (
        jnp.dot(p_ref[...], w_ref[...], preferred_element_type=jnp.float32)
        + b_ref[...])


def _project(pooled, W, b):
    return pl.pallas_call(
        _mm_body,
        out_shape=jax.ShapeDtypeStruct((B, LABEL), jnp.float32),
    )(pooled, W, b.reshape(1, LABEL))


def kernel(input, emb_table, W, b):
    idx = input.astype(jnp.int32).reshape(B * L // CHUNK, CHUNK)
    pooled = _pool_sc(idx, emb_table)
    return _project(pooled, W, b)
